# Initial kernel scaffold; baseline (speedup 1.0000x reference)
import jax
import jax.numpy as jnp
from jax.experimental import pallas as pl


def kernel(x, emb_tim):
    raise NotImplementedError("write your pallas kernel here")


# speedup vs baseline: 6.3445x; 6.3445x over previous
import functools

import jax
import jax.numpy as jnp
from jax import lax
from jax.experimental import pallas as pl
from jax.experimental.pallas import tpu as pltpu
from jax.experimental.pallas import tpu_sc as plsc

_BATCH = 16384
_SEQ = 200
_D = 32
_TOTAL = _BATCH * _SEQ
_NC = 2
_NS = 16
_NW = _NC * _NS
_PER_W = _TOTAL // _NW
_CHUNK = 2048
_NCHUNK = _PER_W // _CHUNK


def _emb_body(table_hbm, idx_hbm, out_hbm, idx_v, rows_v, sem):
    wid = lax.axis_index("s") * _NC + lax.axis_index("c")
    base = wid * _PER_W

    def body(i, carry):
        off = base + i * _CHUNK
        pltpu.sync_copy(idx_hbm.at[pl.ds(off, _CHUNK)], idx_v)
        pltpu.async_copy(table_hbm.at[idx_v], rows_v, sem).wait()
        pltpu.sync_copy(rows_v, out_hbm.at[pl.ds(off, _CHUNK)])
        return carry

    lax.fori_loop(0, _NCHUNK, body, 0)


@jax.jit
def kernel(x, emb_tim):
    idx = x.reshape(-1).astype(jnp.int32)
    mesh = plsc.VectorSubcoreMesh(core_axis_name="c", subcore_axis_name="s")
    run = functools.partial(
        pl.kernel,
        mesh=mesh,
        out_type=jax.ShapeDtypeStruct((_TOTAL, _D), jnp.float32),
        scratch_types=[
            pltpu.VMEM((_CHUNK,), jnp.int32),
            pltpu.VMEM((_CHUNK, _D), jnp.float32),
            pltpu.SemaphoreType.DMA,
        ],
        compiler_params=pltpu.CompilerParams(use_tc_tiling_on_sc=False),
    )(_emb_body)
    out = run(emb_tim, idx)
    return out.reshape(_BATCH, _SEQ, _D)


# speedup vs baseline: 6.4993x; 1.0244x over previous
import functools

import jax
import jax.numpy as jnp
from jax import lax
from jax.experimental import pallas as pl
from jax.experimental.pallas import tpu as pltpu
from jax.experimental.pallas import tpu_sc as plsc

_BATCH = 16384
_SEQ = 200
_D = 32
_TOTAL = _BATCH * _SEQ
_NC = 2
_NS = 16
_NW = _NC * _NS
_PER_W = _TOTAL // _NW
_CHUNK = 1600
_NCHUNK = _PER_W // _CHUNK


def _emb_body(table_hbm, idx_hbm, out_hbm,
              idx0, idx1, rows0, rows1, si0, si1, sg0, sg1, so0, so1):
    wid = lax.axis_index("s") * _NC + lax.axis_index("c")
    base = wid * _PER_W
    idx_v = (idx0, idx1)
    rows_v = (rows0, rows1)
    si = (si0, si1)
    sg = (sg0, sg1)
    so = (so0, so1)

    def start_idx(i, b):
        off = base + i * _CHUNK
        pltpu.async_copy(idx_hbm.at[pl.ds(off, _CHUNK)], idx_v[b], si[b])

    def wait_idx(b):
        pltpu.make_async_copy(
            idx_hbm.at[pl.ds(0, _CHUNK)], idx_v[b], si[b]).wait()

    def start_gather(b):
        pltpu.async_copy(table_hbm.at[idx_v[b]], rows_v[b], sg[b])

    def wait_gather(b):
        pltpu.make_async_copy(
            table_hbm.at[idx_v[b]], rows_v[b], sg[b]).wait()

    def start_out(i, b):
        off = base + i * _CHUNK
        pltpu.async_copy(rows_v[b], out_hbm.at[pl.ds(off, _CHUNK)], so[b])

    def wait_out(b):
        pltpu.make_async_copy(
            rows_v[b], out_hbm.at[pl.ds(0, _CHUNK)], so[b]).wait()

    start_idx(0, 0)
    start_idx(1, 1)
    wait_idx(0)
    start_gather(0)
    wait_idx(1)
    start_gather(1)
    wait_gather(0)
    start_out(0, 0)
    start_idx(2, 0)

    def step_fn(step, carry):
        for b in (0, 1):
            i = 2 * step + b
            wait_idx(b)
            wait_out(b)
            start_gather(b)
            wait_gather(b ^ 1)
            start_out(i - 1, b ^ 1)

            @pl.when(i + 1 < _NCHUNK)
            def _():
                start_idx(i + 1, b ^ 1)
        return carry

    lax.fori_loop(1, _NCHUNK // 2, step_fn, 0)

    b_last = (_NCHUNK - 1) & 1
    wait_gather(b_last)
    start_out(_NCHUNK - 1, b_last)
    wait_out(b_last ^ 1)
    wait_out(b_last)


@jax.jit
def kernel(x, emb_tim):
    idx = x.reshape(-1).astype(jnp.int32)
    mesh = plsc.VectorSubcoreMesh(core_axis_name="c", subcore_axis_name="s")
    run = functools.partial(
        pl.kernel,
        mesh=mesh,
        out_type=jax.ShapeDtypeStruct((_TOTAL, _D), jnp.float32),
        scratch_types=[
            pltpu.VMEM((_CHUNK,), jnp.int32),
            pltpu.VMEM((_CHUNK,), jnp.int32),
            pltpu.VMEM((_CHUNK, _D), jnp.float32),
            pltpu.VMEM((_CHUNK, _D), jnp.float32),
            pltpu.SemaphoreType.DMA,
            pltpu.SemaphoreType.DMA,
            pltpu.SemaphoreType.DMA,
            pltpu.SemaphoreType.DMA,
            pltpu.SemaphoreType.DMA,
            pltpu.SemaphoreType.DMA,
        ],
        compiler_params=pltpu.CompilerParams(use_tc_tiling_on_sc=False),
    )(_emb_body)
    out = run(emb_tim, idx)
    return out.reshape(_BATCH, _SEQ, _D)
